# baseline (speedup 1.0000x reference)
import functools

import jax
import jax.numpy as jnp
from jax import lax
from jax.experimental import pallas as pl
from jax.experimental.pallas import tpu as pltpu
from jax.experimental.pallas import tpu_sc as plsc

N_TOKENS = 16384
N_EMBED = 2048
N_EXPERTS = 64
K_TOP = 8

NC = 2
NS = 16
LANES = 16
NW = NC * NS

MM_BLK = 1024
SPLITS = (12288,)
TC_TAIL = 4096


def _matmul_body(x_ref, w_ref, b_ref, out_ref):
    out_ref[...] = (
        lax.dot_general(
            w_ref[...], x_ref[...],
            (((1,), (1,)), ((), ())),
            preferred_element_type=jnp.float32,
        )
        + b_ref[...]
    )


def _logits_t(x, W, b2, start_blk, nblk):
    return pl.pallas_call(
        _matmul_body,
        grid=(nblk,),
        in_specs=[
            pl.BlockSpec((MM_BLK, N_EMBED), lambda i: (start_blk + i, 0)),
            pl.BlockSpec((N_EXPERTS, N_EMBED), lambda i: (0, 0)),
            pl.BlockSpec((N_EXPERTS, 1), lambda i: (0, 0)),
        ],
        out_specs=pl.BlockSpec((N_EXPERTS, MM_BLK), lambda i: (0, i)),
        out_shape=jax.ShapeDtypeStruct((N_EXPERTS, nblk * MM_BLK), jnp.float32),
    )(x, W, b2)


def _mm_topk_body(x_ref, w_ref, b_ref, idx_ref, gates_ref):
    logits = (
        lax.dot_general(
            w_ref[...], x_ref[...],
            (((1,), (1,)), ((), ())),
            preferred_element_type=jnp.float32,
        )
        + b_ref[...]
    )
    row = lax.broadcasted_iota(jnp.int32, (N_EXPERTS, MM_BLK), 0)
    vals = logits
    tops = []
    for k in range(K_TOP):
        m = jnp.max(vals, axis=0)
        hit = vals == m[None, :]
        idx = jnp.min(jnp.where(hit, row, N_EXPERTS), axis=0)
        tops.append(m)
        idx_ref[k, :] = idx
        vals = jnp.where(row == idx[None, :], -jnp.inf, vals)
    t0 = tops[0]
    exps = [jnp.exp(t - t0) for t in tops]
    total = exps[0]
    for k in range(1, K_TOP):
        total = total + exps[k]
    inv = 1.0 / total
    for k in range(K_TOP):
        gates_ref[k, :] = exps[k] * inv


def _mm_topk(x, W, b2, start_blk, nblk):
    return pl.pallas_call(
        _mm_topk_body,
        grid=(nblk,),
        in_specs=[
            pl.BlockSpec((MM_BLK, N_EMBED), lambda i: (start_blk + i, 0)),
            pl.BlockSpec((N_EXPERTS, N_EMBED), lambda i: (0, 0)),
            pl.BlockSpec((N_EXPERTS, 1), lambda i: (0, 0)),
        ],
        out_specs=[
            pl.BlockSpec((K_TOP, MM_BLK), lambda i: (0, i)),
            pl.BlockSpec((K_TOP, MM_BLK), lambda i: (0, i)),
        ],
        out_shape=[
            jax.ShapeDtypeStruct((K_TOP, nblk * MM_BLK), jnp.int32),
            jax.ShapeDtypeStruct((K_TOP, nblk * MM_BLK), jnp.float32),
        ],
    )(x, W, b2)


def _make_topk_body(tok_w, n_groups):
    def _topk_body(logits_hbm, idx_hbm, gates_hbm, logits_v, idx_v, gates_v):
        wid = lax.axis_index("s") * NC + lax.axis_index("c")
        base = wid * tok_w
        pltpu.sync_copy(logits_hbm.at[:, pl.ds(base, tok_w)], logits_v)

        def group_body(c, _):
            off = c * LANES

            def expert_body(e, carry):
                s = list(carry[:K_TOP])
                ids = list(carry[K_TOP:])
                v = logits_v[e, pl.ds(off, LANES)]
                iv = jnp.full((LANES,), e, dtype=jnp.int32)
                for k in range(K_TOP):
                    m = v > s[k]
                    sv, si = s[k], ids[k]
                    s[k] = jnp.where(m, v, sv)
                    ids[k] = jnp.where(m, iv, si)
                    v = jnp.where(m, sv, v)
                    iv = jnp.where(m, si, iv)
                return tuple(s) + tuple(ids)

            neg = jnp.full((LANES,), -jnp.inf, dtype=jnp.float32)
            zero = jnp.zeros((LANES,), dtype=jnp.int32)
            init = (neg,) * K_TOP + (zero,) * K_TOP
            carry = lax.fori_loop(0, N_EXPERTS, expert_body, init)
            s = carry[:K_TOP]
            ids = carry[K_TOP:]

            exps = [jnp.exp(s[k] - s[0]) for k in range(K_TOP)]
            total = exps[0]
            for k in range(1, K_TOP):
                total = total + exps[k]
            inv = jnp.float32(1.0) / total
            for k in range(K_TOP):
                idx_v[k, pl.ds(off, LANES)] = ids[k]
                gates_v[k, pl.ds(off, LANES)] = exps[k] * inv
            return 0

        lax.fori_loop(0, n_groups, group_body, 0)
        pltpu.sync_copy(idx_v, idx_hbm.at[:, pl.ds(base, tok_w)])
        pltpu.sync_copy(gates_v, gates_hbm.at[:, pl.ds(base, tok_w)])

    return _topk_body


@functools.cache
def _topk_sc(chunk):
    tok_w = chunk // NW
    n_groups = tok_w // LANES
    return functools.partial(
        pl.kernel,
        out_type=(
            jax.ShapeDtypeStruct((K_TOP, chunk), jnp.int32),
            jax.ShapeDtypeStruct((K_TOP, chunk), jnp.float32),
        ),
        mesh=plsc.VectorSubcoreMesh(core_axis_name="c", subcore_axis_name="s",
                                    num_cores=NC, num_subcores=NS),
        scratch_types=[
            pltpu.VMEM((N_EXPERTS, tok_w), jnp.float32),
            pltpu.VMEM((K_TOP, tok_w), jnp.int32),
            pltpu.VMEM((K_TOP, tok_w), jnp.float32),
        ],
    )(_make_topk_body(tok_w, n_groups))


def kernel(x, W, b):
    b2 = b.reshape(N_EXPERTS, 1)
    idx_parts = []
    gate_parts = []
    off = 0
    for chunk in SPLITS:
        logits_t = _logits_t(x, W, b2, off // MM_BLK, chunk // MM_BLK)
        idx_c, gates_c = _topk_sc(chunk)(logits_t)
        idx_parts.append(idx_c.T)
        gate_parts.append(gates_c.T)
        off += chunk
    idx_c, gates_c = _mm_topk(x, W, b2, off // MM_BLK, TC_TAIL // MM_BLK)
    idx_parts.append(idx_c.T)
    gate_parts.append(gates_c.T)
    return (jnp.concatenate(idx_parts, axis=0),
            jnp.concatenate(gate_parts, axis=0))


# speedup vs baseline: 1.0314x; 1.0314x over previous
import functools

import jax
import jax.numpy as jnp
from jax import lax
from jax.experimental import pallas as pl
from jax.experimental.pallas import tpu as pltpu
from jax.experimental.pallas import tpu_sc as plsc

N_TOKENS = 16384
N_EMBED = 2048
N_EXPERTS = 64
K_TOP = 8

NC = 2
NS = 16
LANES = 16
NW = NC * NS

MM_BLK = 1024
SPLITS = (8192,)
TC_TAIL = 8192


def _matmul_body(x_ref, w_ref, b_ref, out_ref):
    out_ref[...] = (
        lax.dot_general(
            w_ref[...], x_ref[...],
            (((1,), (1,)), ((), ())),
            preferred_element_type=jnp.float32,
        )
        + b_ref[...]
    )


def _logits_t(x, W, b2, start_blk, nblk):
    return pl.pallas_call(
        _matmul_body,
        grid=(nblk,),
        in_specs=[
            pl.BlockSpec((MM_BLK, N_EMBED), lambda i: (start_blk + i, 0)),
            pl.BlockSpec((N_EXPERTS, N_EMBED), lambda i: (0, 0)),
            pl.BlockSpec((N_EXPERTS, 1), lambda i: (0, 0)),
        ],
        out_specs=pl.BlockSpec((N_EXPERTS, MM_BLK), lambda i: (0, i)),
        out_shape=jax.ShapeDtypeStruct((N_EXPERTS, nblk * MM_BLK), jnp.float32),
    )(x, W, b2)


def _mm_topk_body(x_ref, w_ref, b_ref, idx_ref, gates_ref):
    logits = (
        lax.dot_general(
            w_ref[...], x_ref[...],
            (((1,), (1,)), ((), ())),
            preferred_element_type=jnp.float32,
        )
        + b_ref[...]
    )
    row = lax.broadcasted_iota(jnp.int32, (N_EXPERTS, MM_BLK), 0)
    vals = logits
    tops = []
    for k in range(K_TOP):
        m = jnp.max(vals, axis=0)
        hit = vals == m[None, :]
        idx = jnp.min(jnp.where(hit, row, N_EXPERTS), axis=0)
        tops.append(m)
        idx_ref[k, :] = idx
        vals = jnp.where(row == idx[None, :], -jnp.inf, vals)
    t0 = tops[0]
    exps = [jnp.exp(t - t0) for t in tops]
    total = exps[0]
    for k in range(1, K_TOP):
        total = total + exps[k]
    inv = 1.0 / total
    for k in range(K_TOP):
        gates_ref[k, :] = exps[k] * inv


def _mm_topk(x, W, b2, start_blk, nblk):
    return pl.pallas_call(
        _mm_topk_body,
        grid=(nblk,),
        in_specs=[
            pl.BlockSpec((MM_BLK, N_EMBED), lambda i: (start_blk + i, 0)),
            pl.BlockSpec((N_EXPERTS, N_EMBED), lambda i: (0, 0)),
            pl.BlockSpec((N_EXPERTS, 1), lambda i: (0, 0)),
        ],
        out_specs=[
            pl.BlockSpec((K_TOP, MM_BLK), lambda i: (0, i)),
            pl.BlockSpec((K_TOP, MM_BLK), lambda i: (0, i)),
        ],
        out_shape=[
            jax.ShapeDtypeStruct((K_TOP, nblk * MM_BLK), jnp.int32),
            jax.ShapeDtypeStruct((K_TOP, nblk * MM_BLK), jnp.float32),
        ],
    )(x, W, b2)


def _make_topk_body(tok_w, n_groups):
    def _topk_body(logits_hbm, idx_hbm, gates_hbm, logits_v, idx_v, gates_v):
        wid = lax.axis_index("s") * NC + lax.axis_index("c")
        base = wid * tok_w
        pltpu.sync_copy(logits_hbm.at[:, pl.ds(base, tok_w)], logits_v)

        def group_body(c, _):
            off = c * LANES

            def expert_body(e, carry):
                s = list(carry[:K_TOP])
                ids = list(carry[K_TOP:])
                v = logits_v[e, pl.ds(off, LANES)]
                iv = jnp.full((LANES,), e, dtype=jnp.int32)
                for k in range(K_TOP):
                    m = v > s[k]
                    sv, si = s[k], ids[k]
                    s[k] = jnp.where(m, v, sv)
                    ids[k] = jnp.where(m, iv, si)
                    v = jnp.where(m, sv, v)
                    iv = jnp.where(m, si, iv)
                return tuple(s) + tuple(ids)

            neg = jnp.full((LANES,), -jnp.inf, dtype=jnp.float32)
            zero = jnp.zeros((LANES,), dtype=jnp.int32)
            init = (neg,) * K_TOP + (zero,) * K_TOP
            carry = lax.fori_loop(0, N_EXPERTS, expert_body, init)
            s = carry[:K_TOP]
            ids = carry[K_TOP:]

            exps = [jnp.exp(s[k] - s[0]) for k in range(K_TOP)]
            total = exps[0]
            for k in range(1, K_TOP):
                total = total + exps[k]
            inv = jnp.float32(1.0) / total
            for k in range(K_TOP):
                idx_v[k, pl.ds(off, LANES)] = ids[k]
                gates_v[k, pl.ds(off, LANES)] = exps[k] * inv
            return 0

        lax.fori_loop(0, n_groups, group_body, 0)
        pltpu.sync_copy(idx_v, idx_hbm.at[:, pl.ds(base, tok_w)])
        pltpu.sync_copy(gates_v, gates_hbm.at[:, pl.ds(base, tok_w)])

    return _topk_body


@functools.cache
def _topk_sc(chunk):
    tok_w = chunk // NW
    n_groups = tok_w // LANES
    return functools.partial(
        pl.kernel,
        out_type=(
            jax.ShapeDtypeStruct((K_TOP, chunk), jnp.int32),
            jax.ShapeDtypeStruct((K_TOP, chunk), jnp.float32),
        ),
        mesh=plsc.VectorSubcoreMesh(core_axis_name="c", subcore_axis_name="s",
                                    num_cores=NC, num_subcores=NS),
        scratch_types=[
            pltpu.VMEM((N_EXPERTS, tok_w), jnp.float32),
            pltpu.VMEM((K_TOP, tok_w), jnp.int32),
            pltpu.VMEM((K_TOP, tok_w), jnp.float32),
        ],
    )(_make_topk_body(tok_w, n_groups))


def kernel(x, W, b):
    b2 = b.reshape(N_EXPERTS, 1)
    idx_parts = []
    gate_parts = []
    off = 0
    for chunk in SPLITS:
        logits_t = _logits_t(x, W, b2, off // MM_BLK, chunk // MM_BLK)
        idx_c, gates_c = _topk_sc(chunk)(logits_t)
        idx_parts.append(idx_c.T)
        gate_parts.append(gates_c.T)
        off += chunk
    idx_c, gates_c = _mm_topk(x, W, b2, off // MM_BLK, TC_TAIL // MM_BLK)
    idx_parts.append(idx_c.T)
    gate_parts.append(gates_c.T)
    return (jnp.concatenate(idx_parts, axis=0),
            jnp.concatenate(gate_parts, axis=0))
